# baseline (device time: 13663 ns/iter reference)
import jax
import jax.numpy as jnp
from jax import lax
from jax.experimental import pallas as pl
from jax.experimental.pallas import tpu as pltpu

N_DEV = 8
ROWS_A = 128


def kernel(x, k):
    b, s, c = x.shape
    taps = k.shape[0]
    halo = taps - 1

    def body(x_ref, k_ref, out_ref, halo_ref, send_sem, recv_sem):
        my_i = lax.axis_index("i")

        credit_sem = pltpu.get_barrier_semaphore()

        @pl.when(my_i > 0)
        def _():
            pl.semaphore_signal(
                credit_sem,
                inc=1,
                device_id=(my_i - 1,),
                device_id_type=pl.DeviceIdType.MESH,
            )

        kv = k_ref[...].astype(jnp.bfloat16)

        def conv(pad, n):
            acc = pad[:, 0:n, :] * kv[0]
            for t in range(1, taps):
                acc = acc + pad[:, t:t + n, :] * kv[t]
            return acc

        silu = lambda a: a * (1.0 / (1.0 + jnp.exp(-a)))

        xa = x_ref[:, pl.ds(0, ROWS_A), :].astype(jnp.bfloat16)
        pad_a = jnp.concatenate(
            [jnp.zeros((b, halo, c), jnp.bfloat16), xa], axis=1
        )
        acc_a = conv(pad_a, ROWS_A)
        out_ref[:, pl.ds(0, ROWS_A), :] = silu(acc_a)

        send = pltpu.make_async_remote_copy(
            src_ref=x_ref.at[:, pl.ds(s - halo, halo), :],
            dst_ref=halo_ref,
            send_sem=send_sem,
            recv_sem=recv_sem,
            device_id=(lax.rem(my_i + 1, N_DEV),),
            device_id_type=pl.DeviceIdType.MESH,
        )

        @pl.when(my_i < N_DEV - 1)
        def _():
            pl.semaphore_wait(credit_sem, 1)
            send.start()

        pad_b = x_ref[
            :, pl.ds(ROWS_A - halo, s - ROWS_A + halo), :
        ].astype(jnp.bfloat16)
        acc_b = conv(pad_b, s - ROWS_A)
        out_ref[:, pl.ds(ROWS_A, s - ROWS_A), :] = silu(acc_b)

        @pl.when(my_i > 0)
        def _():
            send.wait_recv()
            hv = halo_ref[...].astype(jnp.bfloat16)
            rows = []
            for i in range(halo):
                m = kv[0] * hv[:, i, :]
                for t in range(1, halo - i):
                    m = m + kv[t] * hv[:, i + t, :]
                rows.append(m[:, None, :])
            missing = jnp.concatenate(rows, axis=1)
            a3 = acc_a[:, 0:halo, :] + missing
            out_ref[:, pl.ds(0, halo), :] = silu(a3)

        @pl.when(my_i < N_DEV - 1)
        def _():
            send.wait_send()

    return pl.pallas_call(
        body,
        out_shape=jax.ShapeDtypeStruct((b, s, c), jnp.bfloat16),
        in_specs=[
            pl.BlockSpec(memory_space=pltpu.VMEM),
            pl.BlockSpec(memory_space=pltpu.VMEM),
        ],
        out_specs=pl.BlockSpec(memory_space=pltpu.VMEM),
        scratch_shapes=[
            pltpu.VMEM((b, halo, c), x.dtype),
            pltpu.SemaphoreType.DMA,
            pltpu.SemaphoreType.DMA,
        ],
        compiler_params=pltpu.CompilerParams(collective_id=0),
    )(x, k)


# device time: 11491 ns/iter; 1.1890x vs baseline; 1.1890x over previous
import jax
import jax.numpy as jnp
from jax import lax
from jax.experimental import pallas as pl
from jax.experimental.pallas import tpu as pltpu

N_DEV = 8
ROWS_A = 384


def kernel(x, k):
    b, s, c = x.shape
    taps = k.shape[0]
    halo = taps - 1

    def body(x_ref, k_ref, out_ref, halo_ref, send_sem, recv_sem):
        my_i = lax.axis_index("i")

        credit_sem = pltpu.get_barrier_semaphore()

        @pl.when(my_i > 0)
        def _():
            pl.semaphore_signal(
                credit_sem,
                inc=1,
                device_id=(my_i - 1,),
                device_id_type=pl.DeviceIdType.MESH,
            )

        kv = k_ref[...].astype(jnp.bfloat16)

        def conv(pad, n):
            acc = pad[:, 0:n, :] * kv[0]
            for t in range(1, taps):
                acc = acc + pad[:, t:t + n, :] * kv[t]
            return acc

        silu = lambda a: a * (1.0 / (1.0 + jnp.exp(-a)))

        xa = x_ref[:, pl.ds(0, ROWS_A), :].astype(jnp.bfloat16)
        pad_a = jnp.concatenate(
            [jnp.zeros((b, halo, c), jnp.bfloat16), xa], axis=1
        )
        acc_a = conv(pad_a, ROWS_A)
        out_ref[:, pl.ds(0, ROWS_A), :] = silu(acc_a)

        send = pltpu.make_async_remote_copy(
            src_ref=x_ref.at[:, pl.ds(s - halo, halo), :],
            dst_ref=halo_ref,
            send_sem=send_sem,
            recv_sem=recv_sem,
            device_id=(lax.rem(my_i + 1, N_DEV),),
            device_id_type=pl.DeviceIdType.MESH,
        )

        @pl.when(my_i < N_DEV - 1)
        def _():
            pl.semaphore_wait(credit_sem, 1)
            send.start()

        pad_b = x_ref[
            :, pl.ds(ROWS_A - halo, s - ROWS_A + halo), :
        ].astype(jnp.bfloat16)
        acc_b = conv(pad_b, s - ROWS_A)
        out_ref[:, pl.ds(ROWS_A, s - ROWS_A), :] = silu(acc_b)

        @pl.when(my_i > 0)
        def _():
            send.wait_recv()
            hv = halo_ref[...].astype(jnp.bfloat16)
            rows = []
            for i in range(halo):
                m = kv[0] * hv[:, i, :]
                for t in range(1, halo - i):
                    m = m + kv[t] * hv[:, i + t, :]
                rows.append(m[:, None, :])
            missing = jnp.concatenate(rows, axis=1)
            a3 = acc_a[:, 0:halo, :] + missing
            out_ref[:, pl.ds(0, halo), :] = silu(a3)

        @pl.when(my_i < N_DEV - 1)
        def _():
            send.wait_send()

    return pl.pallas_call(
        body,
        out_shape=jax.ShapeDtypeStruct((b, s, c), jnp.bfloat16),
        in_specs=[
            pl.BlockSpec(memory_space=pltpu.VMEM),
            pl.BlockSpec(memory_space=pltpu.VMEM),
        ],
        out_specs=pl.BlockSpec(memory_space=pltpu.VMEM),
        scratch_shapes=[
            pltpu.VMEM((b, halo, c), x.dtype),
            pltpu.SemaphoreType.DMA,
            pltpu.SemaphoreType.DMA,
        ],
        compiler_params=pltpu.CompilerParams(collective_id=0),
    )(x, k)
